# baseline (speedup 1.0000x reference)
import jax
import jax.numpy as jnp
from jax.experimental import pallas as pl
from jax.experimental.pallas import tpu as pltpu

_G = 5


def _dma3(len_ref, r_ref, t_ref, p_ref, ro_ref, to_ref, po_ref,
          sem_r, sem_t, sem_p):
    idx = len_ref[0] - 1
    g = pl.program_id(0)
    Vb = to_ref.shape[0]
    cp_t = pltpu.make_async_copy(
        t_ref.at[idx, pl.ds(g * Vb, Vb)], to_ref, sem_t)
    cp_t.start()

    @pl.when(g == 0)
    def _():
        cp_r = pltpu.make_async_copy(r_ref.at[idx], ro_ref, sem_r)
        cp_p = pltpu.make_async_copy(p_ref.at[idx], po_ref, sem_p)
        cp_r.start()
        cp_p.start()
        cp_r.wait()
        cp_p.wait()

    cp_t.wait()


def kernel(rule_prob, token_prob, reference_prob, length):
    L, B, R = rule_prob.shape
    V = token_prob.shape[2]
    M = reference_prob.shape[2]
    tok_t = token_prob.transpose(0, 2, 1)
    ref_t = reference_prob.transpose(0, 2, 1)
    Vb = V // _G

    r, t_t, p_t = pl.pallas_call(
        _dma3,
        grid=(_G,),
        in_specs=[
            pl.BlockSpec(memory_space=pltpu.SMEM),
            pl.BlockSpec(memory_space=pl.ANY),
            pl.BlockSpec(memory_space=pl.ANY),
            pl.BlockSpec(memory_space=pl.ANY),
        ],
        out_specs=[
            pl.BlockSpec((B, R), lambda g: (0, 0)),
            pl.BlockSpec((Vb, B), lambda g: (g, 0)),
            pl.BlockSpec((M, B), lambda g: (0, 0)),
        ],
        out_shape=[
            jax.ShapeDtypeStruct((B, R), jnp.float32),
            jax.ShapeDtypeStruct((V, B), jnp.float32),
            jax.ShapeDtypeStruct((M, B), jnp.float32),
        ],
        scratch_shapes=[pltpu.SemaphoreType.DMA] * 3,
    )(length, rule_prob, tok_t, ref_t)
    return (r, t_t.T, p_t.T)


# speedup vs baseline: 1.2004x; 1.2004x over previous
import jax
import jax.numpy as jnp
from jax.experimental import pallas as pl
from jax.experimental.pallas import tpu as pltpu

_KC = 5


def _dma_ring(len_ref, r_ref, t_ref, p_ref, ro_ref, to_ref, po_ref,
              rbuf, tbuf, pbuf, sem_ri, sem_ro, sem_pi, sem_po,
              sem_ti0, sem_ti1, sem_to0, sem_to1):
    idx = len_ref[0] - 1
    V = t_ref.shape[1]
    Vb = V // _KC
    sem_ti = [sem_ti0, sem_ti1]
    sem_to = [sem_to0, sem_to1]

    def tok_in(k):
        c = pltpu.make_async_copy(
            t_ref.at[idx, pl.ds(k * Vb, Vb)], tbuf.at[k % 2], sem_ti[k % 2])
        c.start()
        return c

    def tok_out(k):
        c = pltpu.make_async_copy(
            tbuf.at[k % 2], to_ref.at[pl.ds(k * Vb, Vb)], sem_to[k % 2])
        c.start()
        return c

    cp_ri = pltpu.make_async_copy(r_ref.at[idx], rbuf, sem_ri)
    cp_pi = pltpu.make_async_copy(p_ref.at[idx], pbuf, sem_pi)
    in_h = [None] * _KC
    out_h = [None] * _KC
    in_h[0] = tok_in(0)
    cp_ri.start()
    cp_pi.start()
    in_h[1] = tok_in(1)
    for k in range(_KC):
        in_h[k].wait()
        out_h[k] = tok_out(k)
        if k + 2 < _KC:
            out_h[k].wait()
            in_h[k + 2] = tok_in(k + 2)
    cp_ri.wait()
    cp_ro = pltpu.make_async_copy(rbuf, ro_ref, sem_ro)
    cp_ro.start()
    cp_pi.wait()
    cp_po = pltpu.make_async_copy(pbuf, po_ref, sem_po)
    cp_po.start()
    out_h[_KC - 2].wait()
    out_h[_KC - 1].wait()
    cp_ro.wait()
    cp_po.wait()


def kernel(rule_prob, token_prob, reference_prob, length):
    L, B, R = rule_prob.shape
    V = token_prob.shape[2]
    M = reference_prob.shape[2]
    tok_t = token_prob.transpose(0, 2, 1)
    ref_t = reference_prob.transpose(0, 2, 1)
    Vb = V // _KC

    r, t_t, p_t = pl.pallas_call(
        _dma_ring,
        in_specs=[
            pl.BlockSpec(memory_space=pltpu.SMEM),
            pl.BlockSpec(memory_space=pl.ANY),
            pl.BlockSpec(memory_space=pl.ANY),
            pl.BlockSpec(memory_space=pl.ANY),
        ],
        out_specs=[
            pl.BlockSpec(memory_space=pl.ANY),
            pl.BlockSpec(memory_space=pl.ANY),
            pl.BlockSpec(memory_space=pl.ANY),
        ],
        out_shape=[
            jax.ShapeDtypeStruct((B, R), jnp.float32),
            jax.ShapeDtypeStruct((V, B), jnp.float32),
            jax.ShapeDtypeStruct((M, B), jnp.float32),
        ],
        scratch_shapes=[
            pltpu.VMEM((B, R), jnp.float32),
            pltpu.VMEM((2, Vb, B), jnp.float32),
            pltpu.VMEM((M, B), jnp.float32),
        ] + [pltpu.SemaphoreType.DMA] * 8,
    )(length, rule_prob, tok_t, ref_t)
    return (r, t_t.T, p_t.T)


# speedup vs baseline: 1.7210x; 1.4338x over previous
import jax
import jax.numpy as jnp
from jax.experimental import pallas as pl
from jax.experimental.pallas import tpu as pltpu


def _dma6(len_ref, r_ref, t_ref, p_ref, ro_ref, to_ref, po_ref,
          rbuf, tbuf, pbuf, sem_ri, sem_ro, sem_ti, sem_to, sem_pi, sem_po):
    idx = len_ref[0] - 1
    cp_ti = pltpu.make_async_copy(t_ref.at[idx], tbuf, sem_ti)
    cp_ri = pltpu.make_async_copy(r_ref.at[idx], rbuf, sem_ri)
    cp_pi = pltpu.make_async_copy(p_ref.at[idx], pbuf, sem_pi)
    cp_ti.start()
    cp_ri.start()
    cp_pi.start()
    cp_ri.wait()
    cp_ro = pltpu.make_async_copy(rbuf, ro_ref, sem_ro)
    cp_ro.start()
    cp_pi.wait()
    cp_po = pltpu.make_async_copy(pbuf, po_ref, sem_po)
    cp_po.start()
    cp_ti.wait()
    cp_to = pltpu.make_async_copy(tbuf, to_ref, sem_to)
    cp_to.start()
    cp_ro.wait()
    cp_po.wait()
    cp_to.wait()


def kernel(rule_prob, token_prob, reference_prob, length):
    L, B, R = rule_prob.shape
    V = token_prob.shape[2]
    M = reference_prob.shape[2]
    tok_t = token_prob.transpose(0, 2, 1)
    ref_t = reference_prob.transpose(0, 2, 1)

    r, t_t, p_t = pl.pallas_call(
        _dma6,
        in_specs=[
            pl.BlockSpec(memory_space=pltpu.SMEM),
            pl.BlockSpec(memory_space=pl.ANY),
            pl.BlockSpec(memory_space=pl.ANY),
            pl.BlockSpec(memory_space=pl.ANY),
        ],
        out_specs=[
            pl.BlockSpec(memory_space=pl.ANY),
            pl.BlockSpec(memory_space=pl.ANY),
            pl.BlockSpec(memory_space=pl.ANY),
        ],
        out_shape=[
            jax.ShapeDtypeStruct((B, R), jnp.float32),
            jax.ShapeDtypeStruct((V, B), jnp.float32),
            jax.ShapeDtypeStruct((M, B), jnp.float32),
        ],
        scratch_shapes=[
            pltpu.VMEM((B, R), jnp.float32),
            pltpu.VMEM((V, B), jnp.float32),
            pltpu.VMEM((M, B), jnp.float32),
        ] + [pltpu.SemaphoreType.DMA] * 6,
    )(length, rule_prob, tok_t, ref_t)
    return (r, t_t.T, p_t.T)
